# baseline (speedup 1.0000x reference)
import jax
import jax.numpy as jnp
from jax.experimental import pallas as pl

_H = 2


def _qpe_body(map_t, actor_t, light_t, map_pe, actor_pe, light_pe, time_pe,
              pos_m, pos_n, pos_l, pos_t,
              map_o, actor_o, light_o):
    T = actor_t.shape[1]
    Nh = actor_t.shape[2]
    Lh = light_t.shape[2]
    D = map_t.shape[-1]

    map_o[...] = map_t[...] + (map_pe[...] + pos_m[...])[None]

    time_comb = (time_pe[:T] + pos_t[:T]).reshape(1, T, 1, D)
    actor_comb = (actor_pe[...] + pos_n[...]).reshape(1, 1, Nh, D)
    actor_o[...] = actor_t[...] + actor_comb + time_comb

    light_comb = (light_pe[...] + pos_l[...]).reshape(1, 1, Lh, D)
    light_o[...] = light_t[...] + light_comb + time_comb


def kernel(map_token, actor_token, light_token, map_pe_w, actor_pe_w,
           light_pe_w, time_pe_w, pos_enc):
    B, S, D = map_token.shape
    _, T, N, _ = actor_token.shape
    L = light_token.shape[2]
    Sh, Nh, Lh = S // _H, N // _H, L // _H
    Tp = (T + 7) // 8 * 8

    outs = pl.pallas_call(
        _qpe_body,
        grid=(_H, B),
        in_specs=[
            pl.BlockSpec((1, Sh, D), lambda h, b: (b, h, 0)),
            pl.BlockSpec((1, T, Nh, D), lambda h, b: (b, 0, h, 0)),
            pl.BlockSpec((1, T, Lh, D), lambda h, b: (b, 0, h, 0)),
            pl.BlockSpec((Sh, D), lambda h, b: (h, 0)),
            pl.BlockSpec((Nh, D), lambda h, b: (h, 0)),
            pl.BlockSpec((Lh, D), lambda h, b: (h, 0)),
            pl.BlockSpec((Tp, D), lambda h, b: (0, 0)),
            pl.BlockSpec((Sh, D), lambda h, b: (h, 0)),
            pl.BlockSpec((Nh, D), lambda h, b: (h, 0)),
            pl.BlockSpec((Lh, D), lambda h, b: (h, 0)),
            pl.BlockSpec((Tp, D), lambda h, b: (0, 0)),
        ],
        out_specs=[
            pl.BlockSpec((1, Sh, D), lambda h, b: (b, h, 0)),
            pl.BlockSpec((1, T, Nh, D), lambda h, b: (b, 0, h, 0)),
            pl.BlockSpec((1, T, Lh, D), lambda h, b: (b, 0, h, 0)),
        ],
        out_shape=[
            jax.ShapeDtypeStruct((B, S, D), map_token.dtype),
            jax.ShapeDtypeStruct((B, T, N, D), actor_token.dtype),
            jax.ShapeDtypeStruct((B, T, L, D), light_token.dtype),
        ],
    )(map_token, actor_token, light_token, map_pe_w, actor_pe_w,
      light_pe_w, time_pe_w, pos_enc, pos_enc, pos_enc, pos_enc)
    return tuple(outs)
